# baseline (speedup 1.0000x reference)
import functools

import jax
import jax.numpy as jnp
from jax.experimental import pallas as pl
from jax.experimental.pallas import tpu as pltpu

_BM = 1024
_BNK = 1024
_BE = 1024
_BN = 512
_HP = 8


def _proj_kernel(x_ref, w_ref, o_ref):
    o_ref[...] = jnp.dot(x_ref[...], w_ref[...],
                         preferred_element_type=jnp.float32)


def _split_hi_lo(v):
    hi = v.astype(jnp.bfloat16)
    lo = (v - hi.astype(jnp.float32)).astype(jnp.bfloat16)
    return hi, lo


def _gather_kernel(src_ref, h_ref, o_ref):
    k = pl.program_id(1)
    ids = jax.lax.broadcasted_iota(jnp.int32, (_BNK, _BE), 0) + k * _BNK
    onehot_t = (ids == src_ref[0]).astype(jnp.bfloat16)
    h_hi, h_lo = _split_hi_lo(h_ref[...])
    dn = (((0,), (0,)), ((), ()))
    part = (jax.lax.dot_general(onehot_t, h_hi, dn,
                                preferred_element_type=jnp.float32)
            + jax.lax.dot_general(onehot_t, h_lo, dn,
                                  preferred_element_type=jnp.float32))

    @pl.when(k == 0)
    def _():
        o_ref[...] = part

    @pl.when(k > 0)
    def _():
        o_ref[...] += part


def _agg_kernel(dst_ref, h_ref, hsrc_ref, asrc_ref, adst_ref, bexp_ref,
                b_ref, o_ref, den_ref, *, relu_out):
    i = pl.program_id(0)
    j = pl.program_id(1)
    nj = pl.num_programs(1)

    @pl.when(j == 0)
    def _():
        o_ref[...] = jnp.zeros_like(o_ref)
        den_ref[...] = jnp.zeros_like(den_ref)

    ids = jax.lax.broadcasted_iota(jnp.int32, (_BN, _BE), 0) + i * _BN
    hit = ids == dst_ref[0]
    onehot = hit.astype(jnp.float32)
    onehot_b = hit.astype(jnp.bfloat16)

    hsrc = hsrc_ref[...]
    a_src_e = jnp.dot(hsrc, asrc_ref[...],
                      preferred_element_type=jnp.float32)
    a_dst_n = jnp.dot(h_ref[...], adst_ref[...],
                      preferred_element_type=jnp.float32)
    a_dst_e = jax.lax.dot_general(
        onehot, a_dst_n, (((0,), (0,)), ((), ())),
        preferred_element_type=jnp.float32)

    e = a_src_e + a_dst_e
    e = jnp.where(e > 0, e, 0.2 * e)
    p = jnp.exp(e)

    msg = hsrc * jnp.dot(p, bexp_ref[...],
                         preferred_element_type=jnp.float32)
    m_hi, m_lo = _split_hi_lo(msg)
    o_ref[...] += (jnp.dot(onehot_b, m_hi,
                           preferred_element_type=jnp.float32)
                   + jnp.dot(onehot_b, m_lo,
                             preferred_element_type=jnp.float32))
    den_ref[...] += jnp.dot(onehot, p,
                            preferred_element_type=jnp.float32)

    @pl.when(j == nj - 1)
    def _():
        den_x = jnp.dot(den_ref[...], bexp_ref[...],
                        preferred_element_type=jnp.float32)
        res = o_ref[...] / (den_x + 1e-16) + b_ref[...]
        if relu_out:
            res = jnp.maximum(res, 0.0)
        o_ref[...] = res


def _gat_layer(x_p, src_r, dst_r, w, a_src_m, a_dst_m, b_exp, bias,
               relu_out):
    npad, din = x_p.shape
    hc = w.shape[1]
    ne = src_r.shape[0]
    f32 = jnp.float32

    h = pl.pallas_call(
        _proj_kernel,
        grid=(npad // _BM,),
        in_specs=[
            pl.BlockSpec((_BM, din), lambda t: (t, 0)),
            pl.BlockSpec((din, hc), lambda t: (0, 0)),
        ],
        out_specs=pl.BlockSpec((_BM, hc), lambda t: (t, 0)),
        out_shape=jax.ShapeDtypeStruct((npad, hc), f32),
    )(x_p, w)

    hsrc = pl.pallas_call(
        _gather_kernel,
        grid=(ne, npad // _BNK),
        in_specs=[
            pl.BlockSpec((1, 1, _BE), lambda e, k: (e, 0, 0)),
            pl.BlockSpec((_BNK, hc), lambda e, k: (k, 0)),
        ],
        out_specs=pl.BlockSpec((_BE, hc), lambda e, k: (e, 0)),
        out_shape=jax.ShapeDtypeStruct((ne * _BE, hc), f32),
    )(src_r, h)

    out = pl.pallas_call(
        functools.partial(_agg_kernel, relu_out=relu_out),
        grid=(npad // _BN, ne),
        in_specs=[
            pl.BlockSpec((1, 1, _BE), lambda i, j: (j, 0, 0)),
            pl.BlockSpec((_BN, hc), lambda i, j: (i, 0)),
            pl.BlockSpec((_BE, hc), lambda i, j: (j, 0)),
            pl.BlockSpec((hc, _HP), lambda i, j: (0, 0)),
            pl.BlockSpec((hc, _HP), lambda i, j: (0, 0)),
            pl.BlockSpec((_HP, hc), lambda i, j: (0, 0)),
            pl.BlockSpec((1, hc), lambda i, j: (0, 0)),
        ],
        out_specs=pl.BlockSpec((_BN, hc), lambda i, j: (i, 0)),
        out_shape=jax.ShapeDtypeStruct((npad, hc), f32),
        scratch_shapes=[pltpu.VMEM((_BN, _HP), f32)],
    )(dst_r, h, hsrc, a_src_m, a_dst_m, b_exp, bias)

    return out


def _att_mats(att_src, att_dst, heads, out_ch):
    eye_h = jnp.eye(_HP, dtype=jnp.float32)[:heads]
    asrc = (att_src.reshape(heads, out_ch)[:, :, None]
            * eye_h[:, None, :]).reshape(heads * out_ch, _HP)
    adst = (att_dst.reshape(heads, out_ch)[:, :, None]
            * eye_h[:, None, :]).reshape(heads * out_ch, _HP)
    bexp = jnp.repeat(eye_h.T, out_ch, axis=1)
    return asrc, adst, bexp


def kernel(x, edge_index, W1, att_src1, att_dst1, b1, W2, att_src2,
           att_dst2, b2):
    n, d = x.shape
    e = edge_index.shape[1]

    npad = ((n + _BM - 1) // _BM) * _BM
    etot = e + n
    ne = (etot + _BE - 1) // _BE
    epad = ne * _BE

    loop = jnp.arange(n, dtype=edge_index.dtype)
    src = jnp.concatenate([edge_index[0], loop])
    dst = jnp.concatenate([edge_index[1], loop])
    pad_idx = jnp.full((epad - etot,), n, dtype=edge_index.dtype)
    src_r = jnp.concatenate([src, pad_idx]).reshape(ne, 1, _BE)
    dst_r = jnp.concatenate([dst, pad_idx]).reshape(ne, 1, _BE)

    x_p = jnp.pad(x, ((0, npad - n), (0, 0)))

    a1s, a1d, b1e = _att_mats(att_src1, att_dst1, 8, 8)
    a2s, a2d, b2e = _att_mats(att_src2, att_dst2, 1, d)

    h1 = _gat_layer(x_p, src_r, dst_r, W1, a1s, a1d, b1e,
                    b1.reshape(1, -1), relu_out=True)
    out = _gat_layer(h1, src_r, dst_r, W2, a2s, a2d, b2e,
                     b2.reshape(1, -1), relu_out=False)
    return out[:n]


# speedup vs baseline: 1.4390x; 1.4390x over previous
import functools

import jax
import jax.numpy as jnp
from jax.experimental import pallas as pl
from jax.experimental.pallas import tpu as pltpu

_BM = 1024
_BNK = 1024
_BE = 2048
_BN = 1024
_HP = 8


def _proj_kernel(x_ref, w_ref, o_ref):
    o_ref[...] = jnp.dot(x_ref[...], w_ref[...],
                         preferred_element_type=jnp.float32)


def _gather_kernel(src_ref, h_ref, o_ref):
    k = pl.program_id(1)
    ids = jax.lax.broadcasted_iota(jnp.int32, (_BNK, _BE), 0) + k * _BNK
    onehot_t = (ids == src_ref[0]).astype(jnp.float32)
    part = jax.lax.dot_general(
        onehot_t, h_ref[...], (((0,), (0,)), ((), ())),
        preferred_element_type=jnp.float32)

    @pl.when(k == 0)
    def _():
        o_ref[...] = part

    @pl.when(k > 0)
    def _():
        o_ref[...] += part


def _agg_kernel(dst_ref, h_ref, hsrc_ref, asrc_ref, adst_ref, bexp_ref,
                b_ref, o_ref, den_ref, *, relu_out):
    i = pl.program_id(0)
    j = pl.program_id(1)
    nj = pl.num_programs(1)

    @pl.when(j == 0)
    def _():
        o_ref[...] = jnp.zeros_like(o_ref)
        den_ref[...] = jnp.zeros_like(den_ref)

    ids = jax.lax.broadcasted_iota(jnp.int32, (_BN, _BE), 0) + i * _BN
    onehot = (ids == dst_ref[0]).astype(jnp.float32)

    hsrc = hsrc_ref[...]
    a_src_e = jnp.dot(hsrc, asrc_ref[...],
                      preferred_element_type=jnp.float32)
    a_dst_n = jnp.dot(h_ref[...], adst_ref[...],
                      preferred_element_type=jnp.float32)
    a_dst_e = jax.lax.dot_general(
        onehot, a_dst_n, (((0,), (0,)), ((), ())),
        preferred_element_type=jnp.float32)

    e = a_src_e + a_dst_e
    e = jnp.where(e > 0, e, 0.2 * e)
    p = jnp.exp(e)

    msg = hsrc * jnp.dot(p, bexp_ref[...],
                         preferred_element_type=jnp.float32)
    o_ref[...] += jnp.dot(onehot, msg,
                          preferred_element_type=jnp.float32)
    den_ref[...] += jnp.dot(onehot, p,
                            preferred_element_type=jnp.float32)

    @pl.when(j == nj - 1)
    def _():
        den_x = jnp.dot(den_ref[...], bexp_ref[...],
                        preferred_element_type=jnp.float32)
        res = o_ref[...] / (den_x + 1e-16) + b_ref[...]
        if relu_out:
            res = jnp.maximum(res, 0.0)
        o_ref[...] = res


def _gat_layer(x_p, src_r, dst_r, w, a_src_m, a_dst_m, b_exp, bias,
               relu_out):
    npad, din = x_p.shape
    hc = w.shape[1]
    ne = src_r.shape[0]
    f32 = jnp.float32

    h = pl.pallas_call(
        _proj_kernel,
        grid=(npad // _BM,),
        in_specs=[
            pl.BlockSpec((_BM, din), lambda t: (t, 0)),
            pl.BlockSpec((din, hc), lambda t: (0, 0)),
        ],
        out_specs=pl.BlockSpec((_BM, hc), lambda t: (t, 0)),
        out_shape=jax.ShapeDtypeStruct((npad, hc), f32),
    )(x_p, w)

    hsrc = pl.pallas_call(
        _gather_kernel,
        grid=(ne, npad // _BNK),
        in_specs=[
            pl.BlockSpec((1, 1, _BE), lambda e, k: (e, 0, 0)),
            pl.BlockSpec((_BNK, hc), lambda e, k: (k, 0)),
        ],
        out_specs=pl.BlockSpec((_BE, hc), lambda e, k: (e, 0)),
        out_shape=jax.ShapeDtypeStruct((ne * _BE, hc), f32),
    )(src_r, h)

    out = pl.pallas_call(
        functools.partial(_agg_kernel, relu_out=relu_out),
        grid=(npad // _BN, ne),
        in_specs=[
            pl.BlockSpec((1, 1, _BE), lambda i, j: (j, 0, 0)),
            pl.BlockSpec((_BN, hc), lambda i, j: (i, 0)),
            pl.BlockSpec((_BE, hc), lambda i, j: (j, 0)),
            pl.BlockSpec((hc, _HP), lambda i, j: (0, 0)),
            pl.BlockSpec((hc, _HP), lambda i, j: (0, 0)),
            pl.BlockSpec((_HP, hc), lambda i, j: (0, 0)),
            pl.BlockSpec((1, hc), lambda i, j: (0, 0)),
        ],
        out_specs=pl.BlockSpec((_BN, hc), lambda i, j: (i, 0)),
        out_shape=jax.ShapeDtypeStruct((npad, hc), f32),
        scratch_shapes=[pltpu.VMEM((_BN, _HP), f32)],
    )(dst_r, h, hsrc, a_src_m, a_dst_m, b_exp, bias)

    return out


def _att_mats(att_src, att_dst, heads, out_ch):
    eye_h = jnp.eye(_HP, dtype=jnp.float32)[:heads]
    asrc = (att_src.reshape(heads, out_ch)[:, :, None]
            * eye_h[:, None, :]).reshape(heads * out_ch, _HP)
    adst = (att_dst.reshape(heads, out_ch)[:, :, None]
            * eye_h[:, None, :]).reshape(heads * out_ch, _HP)
    bexp = jnp.repeat(eye_h.T, out_ch, axis=1)
    return asrc, adst, bexp


def kernel(x, edge_index, W1, att_src1, att_dst1, b1, W2, att_src2,
           att_dst2, b2):
    n, d = x.shape
    e = edge_index.shape[1]

    npad = ((n + _BM - 1) // _BM) * _BM
    etot = e + n
    ne = (etot + _BE - 1) // _BE
    epad = ne * _BE

    loop = jnp.arange(n, dtype=edge_index.dtype)
    src = jnp.concatenate([edge_index[0], loop])
    dst = jnp.concatenate([edge_index[1], loop])
    pad_idx = jnp.full((epad - etot,), n, dtype=edge_index.dtype)
    src_r = jnp.concatenate([src, pad_idx]).reshape(ne, 1, _BE)
    dst_r = jnp.concatenate([dst, pad_idx]).reshape(ne, 1, _BE)

    x_p = jnp.pad(x, ((0, npad - n), (0, 0)))

    a1s, a1d, b1e = _att_mats(att_src1, att_dst1, 8, 8)
    a2s, a2d, b2e = _att_mats(att_src2, att_dst2, 1, d)

    h1 = _gat_layer(x_p, src_r, dst_r, W1, a1s, a1d, b1e,
                    b1.reshape(1, -1), relu_out=True)
    out = _gat_layer(h1, src_r, dst_r, W2, a2s, a2d, b2e,
                     b2.reshape(1, -1), relu_out=False)
    return out[:n]


# speedup vs baseline: 1.4908x; 1.0360x over previous
import functools

import jax
import jax.numpy as jnp
from jax.experimental import pallas as pl
from jax.experimental.pallas import tpu as pltpu

_BM = 1024
_BNK = 1024
_BE = 4096
_BN = 1024
_HP = 8


def _proj_kernel(x_ref, w_ref, o_ref):
    o_ref[...] = jnp.dot(x_ref[...], w_ref[...],
                         preferred_element_type=jnp.float32)


def _gather_kernel(src_ref, h_ref, o_ref):
    k = pl.program_id(1)
    ids = jax.lax.broadcasted_iota(jnp.int32, (_BNK, _BE), 0) + k * _BNK
    onehot_t = (ids == src_ref[0]).astype(jnp.float32)
    part = jax.lax.dot_general(
        onehot_t, h_ref[...], (((0,), (0,)), ((), ())),
        preferred_element_type=jnp.float32)

    @pl.when(k == 0)
    def _():
        o_ref[...] = part

    @pl.when(k > 0)
    def _():
        o_ref[...] += part


def _agg_kernel(dst_ref, h_ref, hsrc_ref, asrc_ref, adst_ref, bexp_ref,
                b_ref, o_ref, den_ref, *, relu_out):
    i = pl.program_id(0)
    j = pl.program_id(1)
    nj = pl.num_programs(1)

    @pl.when(j == 0)
    def _():
        o_ref[...] = jnp.zeros_like(o_ref)
        den_ref[...] = jnp.zeros_like(den_ref)

    ids = jax.lax.broadcasted_iota(jnp.int32, (_BN, _BE), 0) + i * _BN
    onehot = (ids == dst_ref[0]).astype(jnp.float32)

    hsrc = hsrc_ref[...]
    a_src_e = jnp.dot(hsrc, asrc_ref[...],
                      preferred_element_type=jnp.float32)
    a_dst_n = jnp.dot(h_ref[...], adst_ref[...],
                      preferred_element_type=jnp.float32)
    a_dst_e = jax.lax.dot_general(
        onehot, a_dst_n, (((0,), (0,)), ((), ())),
        preferred_element_type=jnp.float32)

    e = a_src_e + a_dst_e
    e = jnp.where(e > 0, e, 0.2 * e)
    p = jnp.exp(e)

    msg = hsrc * jnp.dot(p, bexp_ref[...],
                         preferred_element_type=jnp.float32)
    o_ref[...] += jnp.dot(onehot, msg,
                          preferred_element_type=jnp.float32)
    den_ref[...] += jnp.dot(onehot, p,
                            preferred_element_type=jnp.float32)

    @pl.when(j == nj - 1)
    def _():
        den_x = jnp.dot(den_ref[...], bexp_ref[...],
                        preferred_element_type=jnp.float32)
        res = o_ref[...] / (den_x + 1e-16) + b_ref[...]
        if relu_out:
            res = jnp.maximum(res, 0.0)
        o_ref[...] = res


def _gat_layer(x_p, src_r, dst_r, w, a_src_m, a_dst_m, b_exp, bias,
               relu_out):
    npad, din = x_p.shape
    hc = w.shape[1]
    ne = src_r.shape[0]
    f32 = jnp.float32

    h = pl.pallas_call(
        _proj_kernel,
        grid=(npad // _BM,),
        in_specs=[
            pl.BlockSpec((_BM, din), lambda t: (t, 0)),
            pl.BlockSpec((din, hc), lambda t: (0, 0)),
        ],
        out_specs=pl.BlockSpec((_BM, hc), lambda t: (t, 0)),
        out_shape=jax.ShapeDtypeStruct((npad, hc), f32),
    )(x_p, w)

    hsrc = pl.pallas_call(
        _gather_kernel,
        grid=(ne, npad // _BNK),
        in_specs=[
            pl.BlockSpec((1, 1, _BE), lambda e, k: (e, 0, 0)),
            pl.BlockSpec((_BNK, hc), lambda e, k: (k, 0)),
        ],
        out_specs=pl.BlockSpec((_BE, hc), lambda e, k: (e, 0)),
        out_shape=jax.ShapeDtypeStruct((ne * _BE, hc), f32),
    )(src_r, h)

    out = pl.pallas_call(
        functools.partial(_agg_kernel, relu_out=relu_out),
        grid=(npad // _BN, ne),
        in_specs=[
            pl.BlockSpec((1, 1, _BE), lambda i, j: (j, 0, 0)),
            pl.BlockSpec((_BN, hc), lambda i, j: (i, 0)),
            pl.BlockSpec((_BE, hc), lambda i, j: (j, 0)),
            pl.BlockSpec((hc, _HP), lambda i, j: (0, 0)),
            pl.BlockSpec((hc, _HP), lambda i, j: (0, 0)),
            pl.BlockSpec((_HP, hc), lambda i, j: (0, 0)),
            pl.BlockSpec((1, hc), lambda i, j: (0, 0)),
        ],
        out_specs=pl.BlockSpec((_BN, hc), lambda i, j: (i, 0)),
        out_shape=jax.ShapeDtypeStruct((npad, hc), f32),
        scratch_shapes=[pltpu.VMEM((_BN, _HP), f32)],
    )(dst_r, h, hsrc, a_src_m, a_dst_m, b_exp, bias)

    return out


def _att_mats(att_src, att_dst, heads, out_ch):
    eye_h = jnp.eye(_HP, dtype=jnp.float32)[:heads]
    asrc = (att_src.reshape(heads, out_ch)[:, :, None]
            * eye_h[:, None, :]).reshape(heads * out_ch, _HP)
    adst = (att_dst.reshape(heads, out_ch)[:, :, None]
            * eye_h[:, None, :]).reshape(heads * out_ch, _HP)
    bexp = jnp.repeat(eye_h.T, out_ch, axis=1)
    return asrc, adst, bexp


def kernel(x, edge_index, W1, att_src1, att_dst1, b1, W2, att_src2,
           att_dst2, b2):
    n, d = x.shape
    e = edge_index.shape[1]

    npad = ((n + _BM - 1) // _BM) * _BM
    etot = e + n
    ne = (etot + _BE - 1) // _BE
    epad = ne * _BE

    loop = jnp.arange(n, dtype=edge_index.dtype)
    src = jnp.concatenate([edge_index[0], loop])
    dst = jnp.concatenate([edge_index[1], loop])
    pad_idx = jnp.full((epad - etot,), n, dtype=edge_index.dtype)
    src_r = jnp.concatenate([src, pad_idx]).reshape(ne, 1, _BE)
    dst_r = jnp.concatenate([dst, pad_idx]).reshape(ne, 1, _BE)

    x_p = jnp.pad(x, ((0, npad - n), (0, 0)))

    a1s, a1d, b1e = _att_mats(att_src1, att_dst1, 8, 8)
    a2s, a2d, b2e = _att_mats(att_src2, att_dst2, 1, d)

    h1 = _gat_layer(x_p, src_r, dst_r, W1, a1s, a1d, b1e,
                    b1.reshape(1, -1), relu_out=True)
    out = _gat_layer(h1, src_r, dst_r, W2, a2s, a2d, b2e,
                     b2.reshape(1, -1), relu_out=False)
    return out[:n]


# speedup vs baseline: 1.5009x; 1.0068x over previous
import functools

import jax
import jax.numpy as jnp
from jax.experimental import pallas as pl
from jax.experimental.pallas import tpu as pltpu

_BM = 1024
_BNK = 2048
_BE = 4096
_BN = 1024
_HP = 8


def _proj_kernel(x_ref, w_ref, o_ref):
    o_ref[...] = jnp.dot(x_ref[...], w_ref[...],
                         preferred_element_type=jnp.float32)


def _gather_kernel(src_ref, h_ref, o_ref):
    k = pl.program_id(1)
    ids = jax.lax.broadcasted_iota(jnp.int32, (_BNK, _BE), 0) + k * _BNK
    onehot_t = (ids == src_ref[0]).astype(jnp.float32)
    part = jax.lax.dot_general(
        onehot_t, h_ref[...], (((0,), (0,)), ((), ())),
        preferred_element_type=jnp.float32)

    @pl.when(k == 0)
    def _():
        o_ref[...] = part

    @pl.when(k > 0)
    def _():
        o_ref[...] += part


def _agg_kernel(dst_ref, h_ref, hsrc_ref, asrc_ref, adst_ref, bexp_ref,
                b_ref, o_ref, den_ref, *, relu_out):
    i = pl.program_id(0)
    j = pl.program_id(1)
    nj = pl.num_programs(1)

    @pl.when(j == 0)
    def _():
        o_ref[...] = jnp.zeros_like(o_ref)
        den_ref[...] = jnp.zeros_like(den_ref)

    ids = jax.lax.broadcasted_iota(jnp.int32, (_BN, _BE), 0) + i * _BN
    onehot = (ids == dst_ref[0]).astype(jnp.float32)

    hsrc = hsrc_ref[...]
    a_src_e = jnp.dot(hsrc, asrc_ref[...],
                      preferred_element_type=jnp.float32)
    a_dst_n = jnp.dot(h_ref[...], adst_ref[...],
                      preferred_element_type=jnp.float32)
    a_dst_e = jax.lax.dot_general(
        onehot, a_dst_n, (((0,), (0,)), ((), ())),
        preferred_element_type=jnp.float32)

    e = a_src_e + a_dst_e
    e = jnp.where(e > 0, e, 0.2 * e)
    p = jnp.exp(e)

    msg = hsrc * jnp.dot(p, bexp_ref[...],
                         preferred_element_type=jnp.float32)
    o_ref[...] += jnp.dot(onehot, msg,
                          preferred_element_type=jnp.float32)
    den_ref[...] += jnp.dot(onehot, p,
                            preferred_element_type=jnp.float32)

    @pl.when(j == nj - 1)
    def _():
        den_x = jnp.dot(den_ref[...], bexp_ref[...],
                        preferred_element_type=jnp.float32)
        res = o_ref[...] / (den_x + 1e-16) + b_ref[...]
        if relu_out:
            res = jnp.maximum(res, 0.0)
        o_ref[...] = res


def _gat_layer(x_p, src_r, dst_r, w, a_src_m, a_dst_m, b_exp, bias,
               relu_out):
    npad, din = x_p.shape
    hc = w.shape[1]
    ne = src_r.shape[0]
    f32 = jnp.float32

    h = pl.pallas_call(
        _proj_kernel,
        grid=(npad // _BM,),
        in_specs=[
            pl.BlockSpec((_BM, din), lambda t: (t, 0)),
            pl.BlockSpec((din, hc), lambda t: (0, 0)),
        ],
        out_specs=pl.BlockSpec((_BM, hc), lambda t: (t, 0)),
        out_shape=jax.ShapeDtypeStruct((npad, hc), f32),
    )(x_p, w)

    hsrc = pl.pallas_call(
        _gather_kernel,
        grid=(ne, npad // _BNK),
        in_specs=[
            pl.BlockSpec((1, 1, _BE), lambda e, k: (e, 0, 0)),
            pl.BlockSpec((_BNK, hc), lambda e, k: (k, 0)),
        ],
        out_specs=pl.BlockSpec((_BE, hc), lambda e, k: (e, 0)),
        out_shape=jax.ShapeDtypeStruct((ne * _BE, hc), f32),
    )(src_r, h)

    out = pl.pallas_call(
        functools.partial(_agg_kernel, relu_out=relu_out),
        grid=(npad // _BN, ne),
        in_specs=[
            pl.BlockSpec((1, 1, _BE), lambda i, j: (j, 0, 0)),
            pl.BlockSpec((_BN, hc), lambda i, j: (i, 0)),
            pl.BlockSpec((_BE, hc), lambda i, j: (j, 0)),
            pl.BlockSpec((hc, _HP), lambda i, j: (0, 0)),
            pl.BlockSpec((hc, _HP), lambda i, j: (0, 0)),
            pl.BlockSpec((_HP, hc), lambda i, j: (0, 0)),
            pl.BlockSpec((1, hc), lambda i, j: (0, 0)),
        ],
        out_specs=pl.BlockSpec((_BN, hc), lambda i, j: (i, 0)),
        out_shape=jax.ShapeDtypeStruct((npad, hc), f32),
        scratch_shapes=[pltpu.VMEM((_BN, _HP), f32)],
    )(dst_r, h, hsrc, a_src_m, a_dst_m, b_exp, bias)

    return out


def _att_mats(att_src, att_dst, heads, out_ch):
    eye_h = jnp.eye(_HP, dtype=jnp.float32)[:heads]
    asrc = (att_src.reshape(heads, out_ch)[:, :, None]
            * eye_h[:, None, :]).reshape(heads * out_ch, _HP)
    adst = (att_dst.reshape(heads, out_ch)[:, :, None]
            * eye_h[:, None, :]).reshape(heads * out_ch, _HP)
    bexp = jnp.repeat(eye_h.T, out_ch, axis=1)
    return asrc, adst, bexp


def kernel(x, edge_index, W1, att_src1, att_dst1, b1, W2, att_src2,
           att_dst2, b2):
    n, d = x.shape
    e = edge_index.shape[1]

    npad = ((n + _BM - 1) // _BM) * _BM
    etot = e + n
    ne = (etot + _BE - 1) // _BE
    epad = ne * _BE

    loop = jnp.arange(n, dtype=edge_index.dtype)
    src = jnp.concatenate([edge_index[0], loop])
    dst = jnp.concatenate([edge_index[1], loop])
    pad_idx = jnp.full((epad - etot,), n, dtype=edge_index.dtype)
    src_r = jnp.concatenate([src, pad_idx]).reshape(ne, 1, _BE)
    dst_r = jnp.concatenate([dst, pad_idx]).reshape(ne, 1, _BE)

    x_p = jnp.pad(x, ((0, npad - n), (0, 0)))

    a1s, a1d, b1e = _att_mats(att_src1, att_dst1, 8, 8)
    a2s, a2d, b2e = _att_mats(att_src2, att_dst2, 1, d)

    h1 = _gat_layer(x_p, src_r, dst_r, W1, a1s, a1d, b1e,
                    b1.reshape(1, -1), relu_out=True)
    out = _gat_layer(h1, src_r, dst_r, W2, a2s, a2d, b2e,
                     b2.reshape(1, -1), relu_out=False)
    return out[:n]
